# baseline (speedup 1.0000x reference)
import functools

import jax
import jax.numpy as jnp
from jax import lax
from jax.experimental import pallas as pl
from jax.experimental.pallas import tpu as pltpu
from jax.experimental.pallas import tpu_sc as plsc

N = 100000
E = 3200000
HID = 32

NC = 2
NS = 16
LANE = 128
JROWS = 3
ROUND_E = JROWS * LANE
ROUNDS = 524
EPT = ROUND_E * ROUNDS
EP = EPT * NS
EPR = EP // LANE
RPT = EPR // NS
NH = N // NC
STRIPE = 3128
ZROWS = NH + NS * STRIPE - NS * STRIPE
ACC_ROWS = NS * STRIPE
DUMMY = ACC_ROWS - 1
LAST_REAL = NH - 15 * STRIPE

_mesh = plsc.VectorSubcoreMesh(core_axis_name="c", subcore_axis_name="s")
_sc_params = pltpu.CompilerParams(use_tc_tiling_on_sc=False)


@functools.partial(
    pl.kernel,
    out_type=[
        jax.ShapeDtypeStruct((N,), jnp.float32),
        jax.ShapeDtypeStruct((NC, EPR, LANE), jnp.int32),
    ],
    mesh=_mesh,
    scratch_types=[
        pltpu.VMEM((JROWS, LANE), jnp.int32),
        pltpu.VMEM((JROWS, LANE), jnp.int32),
        pltpu.VMEM((LANE,), jnp.float32),
        pltpu.VMEM((3136,), jnp.float32),
        pltpu.VMEM_SHARED((ACC_ROWS,), jnp.float32),
        pltpu.SemaphoreType.DMA,
    ],
    compiler_params=_sc_params,
)
def _prep_kernel(dst3, deg_out, ldst_out, dbuf, lbuf, ones, zb, dacc, sem):
  c = lax.axis_index("c")
  s = lax.axis_index("s")
  z16 = jnp.zeros((16,), jnp.float32)

  @pl.loop(0, 3136 // 16)
  def _(i):
    zb[pl.ds(i * 16, 16)] = z16

  @pl.loop(0, LANE // 16)
  def _(i):
    ones[pl.ds(i * 16, 16)] = z16 + 1.0

  pltpu.sync_copy(zb.at[pl.ds(0, STRIPE)], dacc.at[pl.ds(s * STRIPE, STRIPE)])
  plsc.subcore_barrier()

  base = c * NH

  @pl.loop(0, ROUNDS)
  def _(r):
    rbase = s * RPT + r * JROWS
    pltpu.sync_copy(dst3.at[pl.ds(rbase, JROWS), :], dbuf)

    @pl.loop(0, JROWS)
    def _(j):
      @pl.loop(0, LANE // 16)
      def _(k):
        d = dbuf[j, pl.ds(k * 16, 16)]
        ld = d - base
        ok = (ld >= 0) & (ld < NH)
        lbuf[j, pl.ds(k * 16, 16)] = jnp.where(ok, ld, DUMMY)

    pltpu.sync_copy(lbuf, ldst_out.at[c, pl.ds(rbase, JROWS), :])
    descs = [
        pltpu.async_copy(ones, dacc.at[lbuf.at[j]], sem, add=True)
        for j in range(JROWS)
    ]
    for d_ in descs:
      d_.wait()

  plsc.subcore_barrier()
  pltpu.sync_copy(dacc.at[pl.ds(s * STRIPE, STRIPE)], zb.at[pl.ds(0, STRIPE)])

  @pl.when(s < NS - 1)
  def _():
    pltpu.sync_copy(zb.at[pl.ds(0, STRIPE)],
                    deg_out.at[pl.ds(base + s * STRIPE, STRIPE)])

  @pl.when(s == NS - 1)
  def _():
    pltpu.sync_copy(zb.at[pl.ds(0, LAST_REAL)],
                    deg_out.at[pl.ds(base + s * STRIPE, LAST_REAL)])


@functools.partial(
    pl.kernel,
    out_type=jax.ShapeDtypeStruct((N, HID), jnp.float32),
    mesh=_mesh,
    scratch_types=[
        pltpu.VMEM((2, JROWS, LANE), jnp.int32),
        pltpu.VMEM((2, JROWS, LANE), jnp.int32),
        pltpu.VMEM((2, ROUND_E, HID), jnp.float32),
        pltpu.VMEM_SHARED((ACC_ROWS, HID), jnp.float32),
        pltpu.SemaphoreType.DMA,
        pltpu.SemaphoreType.DMA,
        pltpu.SemaphoreType.DMA,
    ],
    compiler_params=_sc_params,
)
def _edge_kernel(y_hbm, src3, ldst3, z_out, sbuf, lbuf, rows, zacc,
                 isem, gsem, ssem):
  c = lax.axis_index("c")
  s = lax.axis_index("s")
  z16 = jnp.zeros((16,), jnp.float32)

  @pl.loop(0, 136)
  def _(i):
    rows[0, i, pl.ds(0, 16)] = z16
    rows[0, i, pl.ds(16, 16)] = z16

  @pl.loop(0, STRIPE // 136)
  def _(i):
    pltpu.sync_copy(rows.at[0, pl.ds(0, 136), :],
                    zacc.at[pl.ds(s * STRIPE + i * 136, 136), :])

  plsc.subcore_barrier()

  def sbuf_start(r, p):
    rbase = s * RPT + r * JROWS
    pltpu.async_copy(src3.at[pl.ds(rbase, JROWS), :], sbuf.at[p], isem)

  def lbuf_start(r, p):
    rbase = s * RPT + r * JROWS
    pltpu.async_copy(ldst3.at[c, pl.ds(rbase, JROWS), :], lbuf.at[p], isem)

  def idx_wait(p):
    pltpu.make_async_copy(src3.at[pl.ds(0, JROWS), :], sbuf.at[p], isem).wait()
    pltpu.make_async_copy(ldst3.at[0, pl.ds(0, JROWS), :], lbuf.at[p],
                          isem).wait()

  def gathers(p):
    gd = [
        pltpu.async_copy(y_hbm.at[sbuf.at[p, j]],
                         rows.at[p, pl.ds(j * LANE, LANE), :], gsem)
        for j in range(JROWS)
    ]
    for d_ in gd:
      d_.wait()

  def scat_start(p):
    for j in range(JROWS):
      pltpu.async_copy(rows.at[p, pl.ds(j * LANE, LANE), :],
                       zacc.at[lbuf.at[p, j]], ssem, add=True)

  def scat_drain(p):
    for j in range(JROWS):
      pltpu.make_async_copy(rows.at[p, pl.ds(j * LANE, LANE), :],
                            zacc.at[lbuf.at[p, j]], ssem).wait()

  sbuf_start(0, 0)
  lbuf_start(0, 0)

  @pl.loop(0, ROUNDS // 2)
  def _(rr):
    for off in (0, 1):
      p = off
      r = rr * 2 + off
      idx_wait(p)
      if off == 0:
        sbuf_start(r + 1, 1)
      else:
        @pl.when(rr < ROUNDS // 2 - 1)
        def _():
          sbuf_start(r + 1, 0)
      gathers(p)
      if off == 0:
        @pl.when(rr > 0)
        def _():
          scat_drain(1)
      else:
        scat_drain(0)
      if off == 0:
        lbuf_start(r + 1, 1)
      else:
        @pl.when(rr < ROUNDS // 2 - 1)
        def _():
          lbuf_start(r + 1, 0)
      scat_start(p)

  scat_drain(1)
  plsc.subcore_barrier()

  out_base = c * NH + s * STRIPE

  @pl.loop(0, 8)
  def _(i):
    pltpu.sync_copy(zacc.at[pl.ds(s * STRIPE + i * ROUND_E, ROUND_E), :],
                    rows.at[0])
    pltpu.sync_copy(rows.at[0],
                    z_out.at[pl.ds(out_base + i * ROUND_E, ROUND_E), :])

  @pl.when(s < NS - 1)
  def _():
    pltpu.sync_copy(zacc.at[pl.ds(s * STRIPE + 3072, STRIPE - 3072), :],
                    rows.at[0, pl.ds(0, STRIPE - 3072), :])
    pltpu.sync_copy(rows.at[0, pl.ds(0, STRIPE - 3072), :],
                    z_out.at[pl.ds(out_base + 3072, STRIPE - 3072), :])

  @pl.when(s == NS - 1)
  def _():
    pltpu.sync_copy(zacc.at[pl.ds(s * STRIPE + 3072, LAST_REAL - 3072), :],
                    rows.at[0, pl.ds(0, LAST_REAL - 3072), :])
    pltpu.sync_copy(rows.at[0, pl.ds(0, LAST_REAL - 3072), :],
                    z_out.at[pl.ds(out_base + 3072, LAST_REAL - 3072), :])


BLK = 2000
GRID = N // BLK
_HIGH = lax.Precision.HIGHEST


def _dot(a, b):
  return lax.dot_general(a, b, (((1,), (0,)), ((), ())),
                         precision=_HIGH, preferred_element_type=jnp.float32)


def _in_matmul_body(x_ref, w_ref, deg_ref, o_ref):
  dinv = lax.rsqrt(deg_ref[...] + 1.0)
  o_ref[...] = dinv * _dot(x_ref[...], w_ref[...])


def _stats_body(z_ref, y_ref, deg_ref, b_ref, o_ref, acc):
  i = pl.program_id(0)
  dinv = lax.rsqrt(deg_ref[...] + 1.0)
  pre = dinv * (z_ref[...] + y_ref[...]) + b_ref[...]
  part = jnp.concatenate(
      [jnp.sum(pre, 0, keepdims=True), jnp.sum(pre * pre, 0, keepdims=True)],
      axis=0)

  @pl.when(i == 0)
  def _():
    acc[...] = part

  @pl.when(i > 0)
  def _():
    acc[...] = acc[...] + part

  @pl.when(i == GRID - 1)
  def _():
    o_ref[...] = acc[...]


def _norm(z_ref, y_ref, deg_ref, st_ref, b_ref, g_ref, be_ref):
  dinv = lax.rsqrt(deg_ref[...] + 1.0)
  pre = dinv * (z_ref[...] + y_ref[...]) + b_ref[...]
  st = st_ref[...]
  mean = st[0:1, :] * (1.0 / N)
  var = st[1:2, :] * (1.0 / N) - mean * mean
  a = g_ref[...] * lax.rsqrt(var + 1e-5)
  cc = be_ref[...] - a * mean
  h = jnp.maximum(a * pre + cc, 0.0)
  return dinv, h


def _fuse_body(z_ref, y_ref, deg_ref, st_ref, b_ref, g_ref, be_ref, w_ref,
               o_ref):
  dinv, h = _norm(z_ref, y_ref, deg_ref, st_ref, b_ref, g_ref, be_ref)
  o_ref[...] = dinv * _dot(h, w_ref[...])


def _head_body(z_ref, y_ref, deg_ref, st_ref, b_ref, g_ref, be_ref, w_ref,
               fcb_ref, o_ref):
  _, h = _norm(z_ref, y_ref, deg_ref, st_ref, b_ref, g_ref, be_ref)
  o_ref[...] = _dot(h, w_ref[...]) + fcb_ref[...]


def _row_spec(cols):
  return pl.BlockSpec((BLK, cols), lambda i: (i, 0))


def _full_spec(shape):
  return pl.BlockSpec(shape, lambda i: tuple(0 for _ in shape))


def _in_matmul(x, w, deg_col):
  return pl.pallas_call(
      _in_matmul_body,
      grid=(GRID,),
      in_specs=[_row_spec(x.shape[1]), _full_spec(w.shape), _row_spec(1)],
      out_specs=_row_spec(HID),
      out_shape=jax.ShapeDtypeStruct((N, HID), jnp.float32),
  )(x, w, deg_col)


def _stats(z, y, deg_col, b2):
  return pl.pallas_call(
      _stats_body,
      grid=(GRID,),
      in_specs=[_row_spec(HID), _row_spec(HID), _row_spec(1),
                _full_spec((1, HID))],
      out_specs=_full_spec((2, HID)),
      out_shape=jax.ShapeDtypeStruct((2, HID), jnp.float32),
      scratch_shapes=[pltpu.VMEM((2, HID), jnp.float32)],
  )(z, y, deg_col, b2)


def _fuse(z, y, deg_col, st, b2, g2, be2, wn):
  return pl.pallas_call(
      _fuse_body,
      grid=(GRID,),
      in_specs=[_row_spec(HID), _row_spec(HID), _row_spec(1),
                _full_spec((2, HID)), _full_spec((1, HID)),
                _full_spec((1, HID)), _full_spec((1, HID)),
                _full_spec((HID, HID))],
      out_specs=_row_spec(HID),
      out_shape=jax.ShapeDtypeStruct((N, HID), jnp.float32),
  )(z, y, deg_col, st, b2, g2, be2, wn)


def _head(z, y, deg_col, st, b2, g2, be2, fcw, fcb2):
  return pl.pallas_call(
      _head_body,
      grid=(GRID,),
      in_specs=[_row_spec(HID), _row_spec(HID), _row_spec(1),
                _full_spec((2, HID)), _full_spec((1, HID)),
                _full_spec((1, HID)), _full_spec((1, HID)),
                _full_spec((HID, 1)), _full_spec((1, 1))],
      out_specs=_row_spec(1),
      out_shape=jax.ShapeDtypeStruct((N, 1), jnp.float32),
  )(z, y, deg_col, st, b2, g2, be2, fcw, fcb2)


def kernel(x, edge_index, edge_attr, W1, b1, g1, be1, W2, b2, g2, be2, W3, b3,
           g3, be3, fcW, fcb):
  del edge_attr
  src = edge_index[0]
  dst = edge_index[1]
  pad = EP - E
  src3 = jnp.concatenate([src, jnp.zeros((pad,), jnp.int32)]).reshape(EPR, LANE)
  dst3 = jnp.concatenate([dst, jnp.full((pad,), -1, jnp.int32)]
                         ).reshape(EPR, LANE)

  deg, ldst3 = _prep_kernel(dst3)
  deg_col = deg.reshape(N, 1)

  y = _in_matmul(x, W1, deg_col)
  params = [(b1, g1, be1), (b2, g2, be2), (b3, g3, be3)]
  mats = [W2, W3]
  for l in range(3):
    z = _edge_kernel(y, src3, ldst3)
    bb, gg, be_ = params[l]
    b2_ = bb.reshape(1, HID)
    g2_ = gg.reshape(1, HID)
    be2_ = be_.reshape(1, HID)
    st = _stats(z, y, deg_col, b2_)
    if l < 2:
      y = _fuse(z, y, deg_col, st, b2_, g2_, be2_, mats[l])
    else:
      out = _head(z, y, deg_col, st, b2_, g2_, be2_, fcW,
                  fcb.reshape(1, 1))
  return out


# speedup vs baseline: 2.3627x; 2.3627x over previous
import functools

import jax
import jax.numpy as jnp
from jax import lax
from jax.experimental import pallas as pl
from jax.experimental.pallas import tpu as pltpu
from jax.experimental.pallas import tpu_sc as plsc

N = 100000
E = 3200000
HID = 32

NC = 2
NS = 16
LANE = 128
JROWS = 3
ROUND_E = JROWS * LANE
ROUNDS = 524
EPT = ROUND_E * ROUNDS
EP = EPT * NS
EPR = EP // LANE
RPT = EPR // NS
NH = N // NC
STRIPE = 3128
ZROWS = NH + NS * STRIPE - NS * STRIPE
ACC_ROWS = NS * STRIPE
DUMMY = ACC_ROWS - 1
LAST_REAL = NH - 15 * STRIPE

_mesh = plsc.VectorSubcoreMesh(core_axis_name="c", subcore_axis_name="s")
_sc_params = pltpu.CompilerParams(use_tc_tiling_on_sc=False)


@functools.partial(
    pl.kernel,
    out_type=[
        jax.ShapeDtypeStruct((N,), jnp.float32),
        jax.ShapeDtypeStruct((NC, EPR, LANE), jnp.int32),
        jax.ShapeDtypeStruct((NC, EPR, LANE), jnp.int32),
    ],
    mesh=_mesh,
    scratch_types=[
        pltpu.VMEM((JROWS, LANE), jnp.int32),
        pltpu.VMEM((JROWS, LANE), jnp.int32),
        pltpu.VMEM((JROWS, LANE), jnp.int32),
        pltpu.VMEM((JROWS, LANE), jnp.int32),
        pltpu.VMEM((LANE,), jnp.float32),
        pltpu.VMEM((3136,), jnp.float32),
        pltpu.VMEM_SHARED((ACC_ROWS,), jnp.float32),
        pltpu.SemaphoreType.DMA,
    ],
    compiler_params=_sc_params,
)
def _prep_kernel(dst3, src3, deg_out, ldst_out, lsrc_out, dbuf, sbuf, lbuf,
                 msbuf, ones, zb, dacc, sem):
  c = lax.axis_index("c")
  s = lax.axis_index("s")
  z16 = jnp.zeros((16,), jnp.float32)

  @pl.loop(0, 3136 // 16)
  def _(i):
    zb[pl.ds(i * 16, 16)] = z16

  @pl.loop(0, LANE // 16)
  def _(i):
    ones[pl.ds(i * 16, 16)] = z16 + 1.0

  pltpu.sync_copy(zb.at[pl.ds(0, STRIPE)], dacc.at[pl.ds(s * STRIPE, STRIPE)])
  plsc.subcore_barrier()

  base = c * NH

  @pl.loop(0, ROUNDS)
  def _(r):
    rbase = s * RPT + r * JROWS
    pltpu.sync_copy((dst3.at[pl.ds(rbase, JROWS), :],
                     src3.at[pl.ds(rbase, JROWS), :]), (dbuf, sbuf))

    @pl.loop(0, JROWS)
    def _(j):
      @pl.loop(0, LANE // 16)
      def _(k):
        d = dbuf[j, pl.ds(k * 16, 16)]
        sv = sbuf[j, pl.ds(k * 16, 16)]
        ld = d - base
        ok = (ld >= 0) & (ld < NH)
        neg1 = jnp.full((16,), -1, jnp.int32)
        lbuf[j, pl.ds(k * 16, 16)] = jnp.where(ok, ld, neg1)
        msbuf[j, pl.ds(k * 16, 16)] = jnp.where(ok, sv, neg1)

    pltpu.sync_copy((lbuf, msbuf),
                    (ldst_out.at[c, pl.ds(rbase, JROWS), :],
                     lsrc_out.at[c, pl.ds(rbase, JROWS), :]))
    descs = [
        pltpu.async_copy(
            ones, dacc.at[plsc.Indices(lbuf.at[j], ignored_value=-1)],
            sem, add=True)
        for j in range(JROWS)
    ]
    for d_ in descs:
      d_.wait()

  plsc.subcore_barrier()
  pltpu.sync_copy(dacc.at[pl.ds(s * STRIPE, STRIPE)], zb.at[pl.ds(0, STRIPE)])

  @pl.when(s < NS - 1)
  def _():
    pltpu.sync_copy(zb.at[pl.ds(0, STRIPE)],
                    deg_out.at[pl.ds(base + s * STRIPE, STRIPE)])

  @pl.when(s == NS - 1)
  def _():
    pltpu.sync_copy(zb.at[pl.ds(0, LAST_REAL)],
                    deg_out.at[pl.ds(base + s * STRIPE, LAST_REAL)])


@functools.partial(
    pl.kernel,
    out_type=jax.ShapeDtypeStruct((N, HID), jnp.float32),
    mesh=_mesh,
    scratch_types=[
        pltpu.VMEM((2, JROWS, LANE), jnp.int32),
        pltpu.VMEM((2, JROWS, LANE), jnp.int32),
        pltpu.VMEM((2, ROUND_E, HID), jnp.float32),
        pltpu.VMEM_SHARED((ACC_ROWS, HID), jnp.float32),
        pltpu.SemaphoreType.DMA,
        pltpu.SemaphoreType.DMA,
        pltpu.SemaphoreType.DMA,
    ],
    compiler_params=_sc_params,
)
def _edge_kernel(y_hbm, lsrc3, ldst3, z_out, sbuf, lbuf, rows, zacc,
                 isem, gsem, ssem):
  c = lax.axis_index("c")
  s = lax.axis_index("s")
  z16 = jnp.zeros((16,), jnp.float32)

  @pl.loop(0, 136)
  def _(i):
    rows[0, i, pl.ds(0, 16)] = z16
    rows[0, i, pl.ds(16, 16)] = z16

  @pl.loop(0, STRIPE // 136)
  def _(i):
    pltpu.sync_copy(rows.at[0, pl.ds(0, 136), :],
                    zacc.at[pl.ds(s * STRIPE + i * 136, 136), :])

  plsc.subcore_barrier()

  def sbuf_start(r, p):
    rbase = s * RPT + r * JROWS
    pltpu.async_copy(lsrc3.at[c, pl.ds(rbase, JROWS), :], sbuf.at[p], isem)

  def lbuf_start(r, p):
    rbase = s * RPT + r * JROWS
    pltpu.async_copy(ldst3.at[c, pl.ds(rbase, JROWS), :], lbuf.at[p], isem)

  def idx_wait(p):
    pltpu.make_async_copy(lsrc3.at[0, pl.ds(0, JROWS), :], sbuf.at[p],
                          isem).wait()
    pltpu.make_async_copy(ldst3.at[0, pl.ds(0, JROWS), :], lbuf.at[p],
                          isem).wait()

  def gathers(p):
    gd = [
        pltpu.async_copy(
            y_hbm.at[plsc.Indices(sbuf.at[p, j], ignored_value=-1)],
            rows.at[p, pl.ds(j * LANE, LANE), :], gsem)
        for j in range(JROWS)
    ]
    for d_ in gd:
      d_.wait()

  def scat_start(p):
    for j in range(JROWS):
      pltpu.async_copy(rows.at[p, pl.ds(j * LANE, LANE), :],
                       zacc.at[plsc.Indices(lbuf.at[p, j], ignored_value=-1)],
                       ssem, add=True)

  def scat_drain(p):
    for j in range(JROWS):
      pltpu.make_async_copy(
          rows.at[p, pl.ds(j * LANE, LANE), :],
          zacc.at[plsc.Indices(lbuf.at[p, j], ignored_value=-1)], ssem).wait()

  sbuf_start(0, 0)
  lbuf_start(0, 0)

  @pl.loop(0, ROUNDS // 2)
  def _(rr):
    for off in (0, 1):
      p = off
      r = rr * 2 + off
      idx_wait(p)
      if off == 0:
        sbuf_start(r + 1, 1)
      else:
        @pl.when(rr < ROUNDS // 2 - 1)
        def _():
          sbuf_start(r + 1, 0)
      gathers(p)
      if off == 0:
        @pl.when(rr > 0)
        def _():
          scat_drain(1)
      else:
        scat_drain(0)
      if off == 0:
        lbuf_start(r + 1, 1)
      else:
        @pl.when(rr < ROUNDS // 2 - 1)
        def _():
          lbuf_start(r + 1, 0)
      scat_start(p)

  scat_drain(1)
  plsc.subcore_barrier()

  out_base = c * NH + s * STRIPE

  @pl.loop(0, 8)
  def _(i):
    pltpu.sync_copy(zacc.at[pl.ds(s * STRIPE + i * ROUND_E, ROUND_E), :],
                    rows.at[0])
    pltpu.sync_copy(rows.at[0],
                    z_out.at[pl.ds(out_base + i * ROUND_E, ROUND_E), :])

  @pl.when(s < NS - 1)
  def _():
    pltpu.sync_copy(zacc.at[pl.ds(s * STRIPE + 3072, STRIPE - 3072), :],
                    rows.at[0, pl.ds(0, STRIPE - 3072), :])
    pltpu.sync_copy(rows.at[0, pl.ds(0, STRIPE - 3072), :],
                    z_out.at[pl.ds(out_base + 3072, STRIPE - 3072), :])

  @pl.when(s == NS - 1)
  def _():
    pltpu.sync_copy(zacc.at[pl.ds(s * STRIPE + 3072, LAST_REAL - 3072), :],
                    rows.at[0, pl.ds(0, LAST_REAL - 3072), :])
    pltpu.sync_copy(rows.at[0, pl.ds(0, LAST_REAL - 3072), :],
                    z_out.at[pl.ds(out_base + 3072, LAST_REAL - 3072), :])


BLK = 2000
GRID = N // BLK
_HIGH = lax.Precision.HIGHEST


def _dot(a, b):
  return lax.dot_general(a, b, (((1,), (0,)), ((), ())),
                         precision=_HIGH, preferred_element_type=jnp.float32)


def _in_matmul_body(x_ref, w_ref, deg_ref, o_ref):
  dinv = lax.rsqrt(deg_ref[...] + 1.0)
  o_ref[...] = dinv * _dot(x_ref[...], w_ref[...])


def _stats_body(z_ref, y_ref, deg_ref, b_ref, o_ref, acc):
  i = pl.program_id(0)
  dinv = lax.rsqrt(deg_ref[...] + 1.0)
  pre = dinv * (z_ref[...] + y_ref[...]) + b_ref[...]
  part = jnp.concatenate(
      [jnp.sum(pre, 0, keepdims=True), jnp.sum(pre * pre, 0, keepdims=True)],
      axis=0)

  @pl.when(i == 0)
  def _():
    acc[...] = part

  @pl.when(i > 0)
  def _():
    acc[...] = acc[...] + part

  @pl.when(i == GRID - 1)
  def _():
    o_ref[...] = acc[...]


def _norm(z_ref, y_ref, deg_ref, st_ref, b_ref, g_ref, be_ref):
  dinv = lax.rsqrt(deg_ref[...] + 1.0)
  pre = dinv * (z_ref[...] + y_ref[...]) + b_ref[...]
  st = st_ref[...]
  mean = st[0:1, :] * (1.0 / N)
  var = st[1:2, :] * (1.0 / N) - mean * mean
  a = g_ref[...] * lax.rsqrt(var + 1e-5)
  cc = be_ref[...] - a * mean
  h = jnp.maximum(a * pre + cc, 0.0)
  return dinv, h


def _fuse_body(z_ref, y_ref, deg_ref, st_ref, b_ref, g_ref, be_ref, w_ref,
               o_ref):
  dinv, h = _norm(z_ref, y_ref, deg_ref, st_ref, b_ref, g_ref, be_ref)
  o_ref[...] = dinv * _dot(h, w_ref[...])


def _head_body(z_ref, y_ref, deg_ref, st_ref, b_ref, g_ref, be_ref, w_ref,
               fcb_ref, o_ref):
  _, h = _norm(z_ref, y_ref, deg_ref, st_ref, b_ref, g_ref, be_ref)
  o_ref[...] = _dot(h, w_ref[...]) + fcb_ref[...]


def _row_spec(cols):
  return pl.BlockSpec((BLK, cols), lambda i: (i, 0))


def _full_spec(shape):
  return pl.BlockSpec(shape, lambda i: tuple(0 for _ in shape))


def _in_matmul(x, w, deg_col):
  return pl.pallas_call(
      _in_matmul_body,
      grid=(GRID,),
      in_specs=[_row_spec(x.shape[1]), _full_spec(w.shape), _row_spec(1)],
      out_specs=_row_spec(HID),
      out_shape=jax.ShapeDtypeStruct((N, HID), jnp.float32),
  )(x, w, deg_col)


def _stats(z, y, deg_col, b2):
  return pl.pallas_call(
      _stats_body,
      grid=(GRID,),
      in_specs=[_row_spec(HID), _row_spec(HID), _row_spec(1),
                _full_spec((1, HID))],
      out_specs=_full_spec((2, HID)),
      out_shape=jax.ShapeDtypeStruct((2, HID), jnp.float32),
      scratch_shapes=[pltpu.VMEM((2, HID), jnp.float32)],
  )(z, y, deg_col, b2)


def _fuse(z, y, deg_col, st, b2, g2, be2, wn):
  return pl.pallas_call(
      _fuse_body,
      grid=(GRID,),
      in_specs=[_row_spec(HID), _row_spec(HID), _row_spec(1),
                _full_spec((2, HID)), _full_spec((1, HID)),
                _full_spec((1, HID)), _full_spec((1, HID)),
                _full_spec((HID, HID))],
      out_specs=_row_spec(HID),
      out_shape=jax.ShapeDtypeStruct((N, HID), jnp.float32),
  )(z, y, deg_col, st, b2, g2, be2, wn)


def _head(z, y, deg_col, st, b2, g2, be2, fcw, fcb2):
  return pl.pallas_call(
      _head_body,
      grid=(GRID,),
      in_specs=[_row_spec(HID), _row_spec(HID), _row_spec(1),
                _full_spec((2, HID)), _full_spec((1, HID)),
                _full_spec((1, HID)), _full_spec((1, HID)),
                _full_spec((HID, 1)), _full_spec((1, 1))],
      out_specs=_row_spec(1),
      out_shape=jax.ShapeDtypeStruct((N, 1), jnp.float32),
  )(z, y, deg_col, st, b2, g2, be2, fcw, fcb2)


def kernel(x, edge_index, edge_attr, W1, b1, g1, be1, W2, b2, g2, be2, W3, b3,
           g3, be3, fcW, fcb):
  del edge_attr
  src = edge_index[0]
  dst = edge_index[1]
  pad = EP - E
  src3 = jnp.concatenate([src, jnp.zeros((pad,), jnp.int32)]).reshape(EPR, LANE)
  dst3 = jnp.concatenate([dst, jnp.full((pad,), -1, jnp.int32)]
                         ).reshape(EPR, LANE)

  deg, ldst3, lsrc3 = _prep_kernel(dst3, src3)
  deg_col = deg.reshape(N, 1)

  y = _in_matmul(x, W1, deg_col)
  params = [(b1, g1, be1), (b2, g2, be2), (b3, g3, be3)]
  mats = [W2, W3]
  for l in range(3):
    z = _edge_kernel(y, lsrc3, ldst3)
    bb, gg, be_ = params[l]
    b2_ = bb.reshape(1, HID)
    g2_ = gg.reshape(1, HID)
    be2_ = be_.reshape(1, HID)
    st = _stats(z, y, deg_col, b2_)
    if l < 2:
      y = _fuse(z, y, deg_col, st, b2_, g2_, be2_, mats[l])
    else:
      out = _head(z, y, deg_col, st, b2_, g2_, be2_, fcW,
                  fcb.reshape(1, 1))
  return out


# speedup vs baseline: 2.6703x; 1.1302x over previous
import functools

import jax
import jax.numpy as jnp
from jax import lax
from jax.experimental import pallas as pl
from jax.experimental.pallas import tpu as pltpu
from jax.experimental.pallas import tpu_sc as plsc

N = 100000
E = 3200000
HID = 32

NC = 2
NS = 16
LANE = 128
JROWS = 2
ROUND_E = JROWS * LANE
ROUNDS = 792
EPT = ROUND_E * ROUNDS
EP = EPT * NS
EPR = EP // LANE
RPT = EPR // NS
JP = 8
RP = EPT // (JP * LANE)
NH = N // NC
STRIPE = 3128
ZROWS = NH + NS * STRIPE - NS * STRIPE
ACC_ROWS = NS * STRIPE
DUMMY = ACC_ROWS - 1
LAST_REAL = NH - 15 * STRIPE

_mesh = plsc.VectorSubcoreMesh(core_axis_name="c", subcore_axis_name="s")
_sc_params = pltpu.CompilerParams(use_tc_tiling_on_sc=False)


@functools.partial(
    pl.kernel,
    out_type=[
        jax.ShapeDtypeStruct((N,), jnp.float32),
        jax.ShapeDtypeStruct((NC, EPR, LANE), jnp.int32),
        jax.ShapeDtypeStruct((NC, EPR, LANE), jnp.int32),
    ],
    mesh=_mesh,
    scratch_types=[
        pltpu.VMEM((JP, LANE), jnp.int32),
        pltpu.VMEM((JP, LANE), jnp.int32),
        pltpu.VMEM((JP, LANE), jnp.int32),
        pltpu.VMEM((JP, LANE), jnp.int32),
        pltpu.VMEM((LANE,), jnp.float32),
        pltpu.VMEM((3136,), jnp.float32),
        pltpu.VMEM_SHARED((ACC_ROWS,), jnp.float32),
        pltpu.SemaphoreType.DMA,
    ],
    compiler_params=_sc_params,
)
def _prep_kernel(dst3, src3, deg_out, ldst_out, lsrc_out, dbuf, sbuf, lbuf,
                 msbuf, ones, zb, dacc, sem):
  c = lax.axis_index("c")
  s = lax.axis_index("s")
  z16 = jnp.zeros((16,), jnp.float32)

  @pl.loop(0, 3136 // 16)
  def _(i):
    zb[pl.ds(i * 16, 16)] = z16

  @pl.loop(0, LANE // 16)
  def _(i):
    ones[pl.ds(i * 16, 16)] = z16 + 1.0

  pltpu.sync_copy(zb.at[pl.ds(0, STRIPE)], dacc.at[pl.ds(s * STRIPE, STRIPE)])
  plsc.subcore_barrier()

  base = c * NH

  @pl.loop(0, RP)
  def _(r):
    rbase = s * RPT + r * JP
    pltpu.sync_copy((dst3.at[pl.ds(rbase, JP), :],
                     src3.at[pl.ds(rbase, JP), :]), (dbuf, sbuf))

    @pl.loop(0, JP)
    def _(j):
      @pl.loop(0, LANE // 16)
      def _(k):
        d = dbuf[j, pl.ds(k * 16, 16)]
        sv = sbuf[j, pl.ds(k * 16, 16)]
        ld = d - base
        ok = (ld >= 0) & (ld < NH)
        neg1 = jnp.full((16,), -1, jnp.int32)
        lbuf[j, pl.ds(k * 16, 16)] = jnp.where(ok, ld, neg1)
        msbuf[j, pl.ds(k * 16, 16)] = jnp.where(ok, sv, neg1)

    pltpu.sync_copy((lbuf, msbuf),
                    (ldst_out.at[c, pl.ds(rbase, JP), :],
                     lsrc_out.at[c, pl.ds(rbase, JP), :]))
    descs = [
        pltpu.async_copy(
            ones, dacc.at[plsc.Indices(lbuf.at[j], ignored_value=-1)],
            sem, add=True)
        for j in range(JP)
    ]
    for d_ in descs:
      d_.wait()

  plsc.subcore_barrier()
  pltpu.sync_copy(dacc.at[pl.ds(s * STRIPE, STRIPE)], zb.at[pl.ds(0, STRIPE)])

  @pl.when(s < NS - 1)
  def _():
    pltpu.sync_copy(zb.at[pl.ds(0, STRIPE)],
                    deg_out.at[pl.ds(base + s * STRIPE, STRIPE)])

  @pl.when(s == NS - 1)
  def _():
    pltpu.sync_copy(zb.at[pl.ds(0, LAST_REAL)],
                    deg_out.at[pl.ds(base + s * STRIPE, LAST_REAL)])


@functools.partial(
    pl.kernel,
    out_type=jax.ShapeDtypeStruct((N, HID), jnp.float32),
    mesh=_mesh,
    scratch_types=[
        pltpu.VMEM((3, JROWS, LANE), jnp.int32),
        pltpu.VMEM((4, JROWS, LANE), jnp.int32),
        pltpu.VMEM((3, ROUND_E, HID), jnp.float32),
        pltpu.VMEM_SHARED((ACC_ROWS, HID), jnp.float32),
        pltpu.SemaphoreType.DMA,
        pltpu.SemaphoreType.DMA,
        pltpu.SemaphoreType.DMA,
    ],
    compiler_params=_sc_params,
)
def _edge_kernel(y_hbm, lsrc3, ldst3, z_out, sbuf, lbuf, rows, zacc,
                 isem, gsem, ssem):
  c = lax.axis_index("c")
  s = lax.axis_index("s")
  z16 = jnp.zeros((16,), jnp.float32)

  @pl.loop(0, 136)
  def _(i):
    rows[0, i, pl.ds(0, 16)] = z16
    rows[0, i, pl.ds(16, 16)] = z16

  @pl.loop(0, STRIPE // 136)
  def _(i):
    pltpu.sync_copy(rows.at[0, pl.ds(0, 136), :],
                    zacc.at[pl.ds(s * STRIPE + i * 136, 136), :])

  plsc.subcore_barrier()

  def idx_start(r, ps, pl_):
    rbase = s * RPT + r * JROWS
    pltpu.async_copy(lsrc3.at[c, pl.ds(rbase, JROWS), :], sbuf.at[ps], isem)
    pltpu.async_copy(ldst3.at[c, pl.ds(rbase, JROWS), :], lbuf.at[pl_], isem)

  def idx_wait(ps, pl_):
    pltpu.make_async_copy(lsrc3.at[0, pl.ds(0, JROWS), :], sbuf.at[ps],
                          isem).wait()
    pltpu.make_async_copy(ldst3.at[0, pl.ds(0, JROWS), :], lbuf.at[pl_],
                          isem).wait()

  def gat_start(p):
    for j in range(JROWS):
      pltpu.async_copy(
          y_hbm.at[plsc.Indices(sbuf.at[p, j], ignored_value=-1)],
          rows.at[p, pl.ds(j * LANE, LANE), :], gsem)

  def gat_wait(p):
    for j in range(JROWS):
      pltpu.make_async_copy(
          y_hbm.at[plsc.Indices(sbuf.at[p, j], ignored_value=-1)],
          rows.at[p, pl.ds(j * LANE, LANE), :], gsem).wait()

  def scat_start(pr, pi):
    for j in range(JROWS):
      pltpu.async_copy(rows.at[pr, pl.ds(j * LANE, LANE), :],
                       zacc.at[plsc.Indices(lbuf.at[pi, j], ignored_value=-1)],
                       ssem, add=True)

  def scat_drain(pr, pi):
    for j in range(JROWS):
      pltpu.make_async_copy(
          rows.at[pr, pl.ds(j * LANE, LANE), :],
          zacc.at[plsc.Indices(lbuf.at[pi, j], ignored_value=-1)],
          ssem).wait()

  idx_start(0, 0, 0)
  idx_wait(0, 0)
  gat_start(0)
  idx_start(1, 1, 1)

  UN = 12
  NIT = ROUNDS // UN

  @pl.loop(0, NIT)
  def _(it):
    for u in range(UN):
      r = it * UN + u
      pr, pi = u % 3, u % 4
      pr1, pi1 = (u + 1) % 3, (u + 1) % 4
      pr2, pi2 = (u + 2) % 3, (u + 2) % 4
      gat_wait(pr)
      scat_start(pr, pi)
      if u >= 2:
        scat_drain((u - 2) % 3, (u - 2) % 4)
      else:
        @pl.when(it > 0)
        def _():
          scat_drain((u - 2) % 3, (u - 2) % 4)
      if u < UN - 2:
        idx_wait(pr1, pi1)
        gat_start(pr1)
        idx_start(r + 2, pr2, pi2)
      elif u == UN - 2:
        idx_wait(pr1, pi1)
        gat_start(pr1)

        @pl.when(it < NIT - 1)
        def _():
          idx_start(r + 2, pr2, pi2)
      else:
        @pl.when(it < NIT - 1)
        def _():
          idx_wait(pr1, pi1)
          gat_start(pr1)
          idx_start(r + 2, pr2, pi2)

  scat_drain((ROUNDS - 2) % 3, (ROUNDS - 2) % 4)
  scat_drain((ROUNDS - 1) % 3, (ROUNDS - 1) % 4)
  plsc.subcore_barrier()

  out_base = c * NH + s * STRIPE

  @pl.loop(0, 3072 // ROUND_E)
  def _(i):
    pltpu.sync_copy(zacc.at[pl.ds(s * STRIPE + i * ROUND_E, ROUND_E), :],
                    rows.at[0])
    pltpu.sync_copy(rows.at[0],
                    z_out.at[pl.ds(out_base + i * ROUND_E, ROUND_E), :])

  @pl.when(s < NS - 1)
  def _():
    pltpu.sync_copy(zacc.at[pl.ds(s * STRIPE + 3072, STRIPE - 3072), :],
                    rows.at[0, pl.ds(0, STRIPE - 3072), :])
    pltpu.sync_copy(rows.at[0, pl.ds(0, STRIPE - 3072), :],
                    z_out.at[pl.ds(out_base + 3072, STRIPE - 3072), :])

  @pl.when(s == NS - 1)
  def _():
    pltpu.sync_copy(zacc.at[pl.ds(s * STRIPE + 3072, LAST_REAL - 3072), :],
                    rows.at[0, pl.ds(0, LAST_REAL - 3072), :])
    pltpu.sync_copy(rows.at[0, pl.ds(0, LAST_REAL - 3072), :],
                    z_out.at[pl.ds(out_base + 3072, LAST_REAL - 3072), :])


BLK = 2000
GRID = N // BLK
_HIGH = lax.Precision.HIGHEST


def _dot(a, b):
  return lax.dot_general(a, b, (((1,), (0,)), ((), ())),
                         precision=_HIGH, preferred_element_type=jnp.float32)


def _in_matmul_body(x_ref, w_ref, deg_ref, o_ref):
  dinv = lax.rsqrt(deg_ref[...] + 1.0)
  o_ref[...] = dinv * _dot(x_ref[...], w_ref[...])


def _pre_block(z_ref, y_ref, deg_ref, b_ref):
  dinv = lax.rsqrt(deg_ref[...] + 1.0)
  return dinv, dinv * (z_ref[...] + y_ref[...]) + b_ref[...]


def _norm_block(pre, acc, g_ref, be_ref):
  st = acc[...]
  mean = st[0:1, :] * (1.0 / N)
  var = st[1:2, :] * (1.0 / N) - mean * mean
  a = g_ref[...] * lax.rsqrt(var + 1e-5)
  cc = be_ref[...] - a * mean
  return jnp.maximum(a * pre + cc, 0.0)


def _fuse_body(z_ref, y_ref, deg_ref, b_ref, g_ref, be_ref, w_ref, o_ref,
               acc):
  ph = pl.program_id(0)
  i = pl.program_id(1)
  dinv, pre = _pre_block(z_ref, y_ref, deg_ref, b_ref)

  @pl.when(ph == 0)
  def _():
    part = jnp.concatenate(
        [jnp.sum(pre, 0, keepdims=True),
         jnp.sum(pre * pre, 0, keepdims=True)], axis=0)

    @pl.when(i == 0)
    def _():
      acc[...] = part

    @pl.when(i > 0)
    def _():
      acc[...] = acc[...] + part

  @pl.when(ph == 1)
  def _():
    h = _norm_block(pre, acc, g_ref, be_ref)
    o_ref[...] = dinv * _dot(h, w_ref[...])


def _head_body(z_ref, y_ref, deg_ref, b_ref, g_ref, be_ref, w_ref, fcb_ref,
               o_ref, acc):
  ph = pl.program_id(0)
  i = pl.program_id(1)
  _, pre = _pre_block(z_ref, y_ref, deg_ref, b_ref)

  @pl.when(ph == 0)
  def _():
    part = jnp.concatenate(
        [jnp.sum(pre, 0, keepdims=True),
         jnp.sum(pre * pre, 0, keepdims=True)], axis=0)

    @pl.when(i == 0)
    def _():
      acc[...] = part

    @pl.when(i > 0)
    def _():
      acc[...] = acc[...] + part

  @pl.when(ph == 1)
  def _():
    h = _norm_block(pre, acc, g_ref, be_ref)
    o_ref[...] = _dot(h, w_ref[...]) + fcb_ref[...]


def _row_spec(cols):
  return pl.BlockSpec((BLK, cols), lambda i: (i, 0))


def _row_spec2(cols):
  return pl.BlockSpec((BLK, cols), lambda p, i: (i, 0))


def _full_spec(shape):
  return pl.BlockSpec(shape, lambda i: tuple(0 for _ in shape))


def _full_spec2(shape):
  return pl.BlockSpec(shape, lambda p, i: tuple(0 for _ in shape))


def _in_matmul(x, w, deg_col):
  return pl.pallas_call(
      _in_matmul_body,
      grid=(GRID,),
      in_specs=[_row_spec(x.shape[1]), _full_spec(w.shape), _row_spec(1)],
      out_specs=_row_spec(HID),
      out_shape=jax.ShapeDtypeStruct((N, HID), jnp.float32),
  )(x, w, deg_col)


def _fuse(z, y, deg_col, b2, g2, be2, wn):
  return pl.pallas_call(
      _fuse_body,
      grid=(2, GRID),
      in_specs=[_row_spec2(HID), _row_spec2(HID), _row_spec2(1),
                _full_spec2((1, HID)), _full_spec2((1, HID)),
                _full_spec2((1, HID)), _full_spec2((HID, HID))],
      out_specs=_row_spec2(HID),
      out_shape=jax.ShapeDtypeStruct((N, HID), jnp.float32),
      scratch_shapes=[pltpu.VMEM((2, HID), jnp.float32)],
  )(z, y, deg_col, b2, g2, be2, wn)


def _head(z, y, deg_col, b2, g2, be2, fcw, fcb2):
  return pl.pallas_call(
      _head_body,
      grid=(2, GRID),
      in_specs=[_row_spec2(HID), _row_spec2(HID), _row_spec2(1),
                _full_spec2((1, HID)), _full_spec2((1, HID)),
                _full_spec2((1, HID)), _full_spec2((HID, 1)),
                _full_spec2((1, 1))],
      out_specs=_row_spec2(1),
      out_shape=jax.ShapeDtypeStruct((N, 1), jnp.float32),
      scratch_shapes=[pltpu.VMEM((2, HID), jnp.float32)],
  )(z, y, deg_col, b2, g2, be2, fcw, fcb2)


def kernel(x, edge_index, edge_attr, W1, b1, g1, be1, W2, b2, g2, be2, W3, b3,
           g3, be3, fcW, fcb):
  del edge_attr
  src = edge_index[0]
  dst = edge_index[1]
  pad = EP - E
  src3 = jnp.concatenate([src, jnp.zeros((pad,), jnp.int32)]).reshape(EPR, LANE)
  dst3 = jnp.concatenate([dst, jnp.full((pad,), -1, jnp.int32)]
                         ).reshape(EPR, LANE)

  deg, ldst3, lsrc3 = _prep_kernel(dst3, src3)
  deg_col = deg.reshape(N, 1)

  y = _in_matmul(x, W1, deg_col)
  params = [(b1, g1, be1), (b2, g2, be2), (b3, g3, be3)]
  mats = [W2, W3]
  for l in range(3):
    z = _edge_kernel(y, lsrc3, ldst3)
    bb, gg, be_ = params[l]
    b2_ = bb.reshape(1, HID)
    g2_ = gg.reshape(1, HID)
    be2_ = be_.reshape(1, HID)
    if l < 2:
      y = _fuse(z, y, deg_col, b2_, g2_, be2_, mats[l])
    else:
      out = _head(z, y, deg_col, b2_, g2_, be2_, fcW, fcb.reshape(1, 1))
  return out


# speedup vs baseline: 2.7277x; 1.0215x over previous
import functools

import jax
import jax.numpy as jnp
from jax import lax
from jax.experimental import pallas as pl
from jax.experimental.pallas import tpu as pltpu
from jax.experimental.pallas import tpu_sc as plsc

N = 100000
E = 3200000
HID = 32

NC = 2
NS = 16
LANE = 128
JROWS = 2
ROUND_E = JROWS * LANE
ROUNDS = 792
EPT = ROUND_E * ROUNDS
EP = EPT * NS
EPR = EP // LANE
RPT = EPR // NS
JP = 8
RP = EPT // (JP * LANE)
NH = N // NC
STRIPE = 3128
ZROWS = NH + NS * STRIPE - NS * STRIPE
ACC_ROWS = NS * STRIPE
DUMMY = ACC_ROWS - 1
LAST_REAL = NH - 15 * STRIPE

_mesh = plsc.VectorSubcoreMesh(core_axis_name="c", subcore_axis_name="s")
_sc_params = pltpu.CompilerParams(use_tc_tiling_on_sc=False)


@functools.partial(
    pl.kernel,
    out_type=[
        jax.ShapeDtypeStruct((N,), jnp.float32),
        jax.ShapeDtypeStruct((NC, EPR, LANE), jnp.int32),
        jax.ShapeDtypeStruct((NC, EPR, LANE), jnp.int32),
    ],
    mesh=_mesh,
    scratch_types=[
        pltpu.VMEM((JP, LANE), jnp.int32),
        pltpu.VMEM((JP, LANE), jnp.int32),
        pltpu.VMEM((JP, LANE), jnp.int32),
        pltpu.VMEM((JP, LANE), jnp.int32),
        pltpu.VMEM((LANE,), jnp.float32),
        pltpu.VMEM((3136,), jnp.float32),
        pltpu.VMEM_SHARED((ACC_ROWS,), jnp.float32),
        pltpu.SemaphoreType.DMA,
    ],
    compiler_params=_sc_params,
)
def _prep_kernel(dst3, src3, deg_out, ldst_out, lsrc_out, dbuf, sbuf, lbuf,
                 msbuf, ones, zb, dacc, sem):
  c = lax.axis_index("c")
  s = lax.axis_index("s")
  z16 = jnp.zeros((16,), jnp.float32)

  @pl.loop(0, 3136 // 16)
  def _(i):
    zb[pl.ds(i * 16, 16)] = z16

  @pl.loop(0, LANE // 16)
  def _(i):
    ones[pl.ds(i * 16, 16)] = z16 + 1.0

  pltpu.sync_copy(zb.at[pl.ds(0, STRIPE)], dacc.at[pl.ds(s * STRIPE, STRIPE)])
  plsc.subcore_barrier()

  base = c * NH

  @pl.loop(0, RP)
  def _(r):
    rbase = s * RPT + r * JP
    pltpu.sync_copy((dst3.at[pl.ds(rbase, JP), :],
                     src3.at[pl.ds(rbase, JP), :]), (dbuf, sbuf))

    @pl.loop(0, JP)
    def _(j):
      @pl.loop(0, LANE // 16)
      def _(k):
        d = dbuf[j, pl.ds(k * 16, 16)]
        sv = sbuf[j, pl.ds(k * 16, 16)]
        ld = d - base
        ok = (ld >= 0) & (ld < NH)
        neg1 = jnp.full((16,), -1, jnp.int32)
        lbuf[j, pl.ds(k * 16, 16)] = jnp.where(ok, ld, neg1)
        msbuf[j, pl.ds(k * 16, 16)] = jnp.where(ok, sv, neg1)

    pltpu.sync_copy((lbuf, msbuf),
                    (ldst_out.at[c, pl.ds(rbase, JP), :],
                     lsrc_out.at[c, pl.ds(rbase, JP), :]))
    descs = [
        pltpu.async_copy(
            ones, dacc.at[plsc.Indices(lbuf.at[j], ignored_value=-1)],
            sem, add=True)
        for j in range(JP)
    ]
    for d_ in descs:
      d_.wait()

  plsc.subcore_barrier()
  pltpu.sync_copy(dacc.at[pl.ds(s * STRIPE, STRIPE)], zb.at[pl.ds(0, STRIPE)])

  @pl.when(s < NS - 1)
  def _():
    pltpu.sync_copy(zb.at[pl.ds(0, STRIPE)],
                    deg_out.at[pl.ds(base + s * STRIPE, STRIPE)])

  @pl.when(s == NS - 1)
  def _():
    pltpu.sync_copy(zb.at[pl.ds(0, LAST_REAL)],
                    deg_out.at[pl.ds(base + s * STRIPE, LAST_REAL)])


@functools.partial(
    pl.kernel,
    out_type=jax.ShapeDtypeStruct((N, HID), jnp.float32),
    mesh=_mesh,
    scratch_types=[
        pltpu.VMEM((3, JROWS, LANE), jnp.int32),
        pltpu.VMEM((4, JROWS, LANE), jnp.int32),
        pltpu.VMEM((3, ROUND_E, HID), jnp.float32),
        pltpu.VMEM_SHARED((ACC_ROWS, HID), jnp.float32),
        pltpu.SemaphoreType.DMA,
        pltpu.SemaphoreType.DMA,
        pltpu.SemaphoreType.DMA,
    ],
    compiler_params=_sc_params,
)
def _edge_kernel(y_hbm, lsrc3, ldst3, z_out, sbuf, lbuf, rows, zacc,
                 isem, gsem, ssem):
  c = lax.axis_index("c")
  s = lax.axis_index("s")
  z16 = jnp.zeros((16,), jnp.float32)

  @pl.loop(0, 136)
  def _(i):
    rows[0, i, pl.ds(0, 16)] = z16
    rows[0, i, pl.ds(16, 16)] = z16

  @pl.loop(0, STRIPE // 136)
  def _(i):
    pltpu.sync_copy(rows.at[0, pl.ds(0, 136), :],
                    zacc.at[pl.ds(s * STRIPE + i * 136, 136), :])

  plsc.subcore_barrier()

  def idx_start(r, ps, pl_):
    rbase = s * RPT + r * JROWS
    pltpu.async_copy(lsrc3.at[c, pl.ds(rbase, JROWS), :], sbuf.at[ps], isem)
    pltpu.async_copy(ldst3.at[c, pl.ds(rbase, JROWS), :], lbuf.at[pl_], isem)

  def idx_wait(ps, pl_):
    pltpu.make_async_copy(lsrc3.at[0, pl.ds(0, JROWS), :], sbuf.at[ps],
                          isem).wait()
    pltpu.make_async_copy(ldst3.at[0, pl.ds(0, JROWS), :], lbuf.at[pl_],
                          isem).wait()

  def gat_start(p):
    for j in range(JROWS):
      pltpu.async_copy(
          y_hbm.at[plsc.Indices(sbuf.at[p, j], ignored_value=-1)],
          rows.at[p, pl.ds(j * LANE, LANE), :], gsem)

  def gat_wait(p):
    for j in range(JROWS):
      pltpu.make_async_copy(
          y_hbm.at[plsc.Indices(sbuf.at[p, j], ignored_value=-1)],
          rows.at[p, pl.ds(j * LANE, LANE), :], gsem).wait()

  def scat_start(pr, pi):
    for j in range(JROWS):
      pltpu.async_copy(rows.at[pr, pl.ds(j * LANE, LANE), :],
                       zacc.at[plsc.Indices(lbuf.at[pi, j], ignored_value=-1)],
                       ssem, add=True)

  def scat_drain(pr, pi):
    for j in range(JROWS):
      pltpu.make_async_copy(
          rows.at[pr, pl.ds(j * LANE, LANE), :],
          zacc.at[plsc.Indices(lbuf.at[pi, j], ignored_value=-1)],
          ssem).wait()

  idx_start(0, 0, 0)
  idx_wait(0, 0)
  gat_start(0)
  idx_start(1, 1, 1)

  UN = 12
  NIT = ROUNDS // UN

  @pl.loop(0, NIT)
  def _(it):
    for u in range(UN):
      r = it * UN + u
      pr, pi = u % 3, u % 4
      pr1, pi1 = (u + 1) % 3, (u + 1) % 4
      pr2, pi2 = (u + 2) % 3, (u + 2) % 4
      gat_wait(pr)
      scat_start(pr, pi)
      if u >= 2:
        scat_drain((u - 2) % 3, (u - 2) % 4)
      else:
        @pl.when(it > 0)
        def _():
          scat_drain((u - 2) % 3, (u - 2) % 4)
      if u < UN - 2:
        idx_wait(pr1, pi1)
        gat_start(pr1)
        idx_start(r + 2, pr2, pi2)
      elif u == UN - 2:
        idx_wait(pr1, pi1)
        gat_start(pr1)

        @pl.when(it < NIT - 1)
        def _():
          idx_start(r + 2, pr2, pi2)
      else:
        @pl.when(it < NIT - 1)
        def _():
          idx_wait(pr1, pi1)
          gat_start(pr1)
          idx_start(r + 2, pr2, pi2)

  scat_drain((ROUNDS - 2) % 3, (ROUNDS - 2) % 4)
  scat_drain((ROUNDS - 1) % 3, (ROUNDS - 1) % 4)
  plsc.subcore_barrier()

  out_base = c * NH + s * STRIPE

  @pl.loop(0, 3072 // ROUND_E)
  def _(i):
    pltpu.sync_copy(zacc.at[pl.ds(s * STRIPE + i * ROUND_E, ROUND_E), :],
                    rows.at[0])
    pltpu.sync_copy(rows.at[0],
                    z_out.at[pl.ds(out_base + i * ROUND_E, ROUND_E), :])

  @pl.when(s < NS - 1)
  def _():
    pltpu.sync_copy(zacc.at[pl.ds(s * STRIPE + 3072, STRIPE - 3072), :],
                    rows.at[0, pl.ds(0, STRIPE - 3072), :])
    pltpu.sync_copy(rows.at[0, pl.ds(0, STRIPE - 3072), :],
                    z_out.at[pl.ds(out_base + 3072, STRIPE - 3072), :])

  @pl.when(s == NS - 1)
  def _():
    pltpu.sync_copy(zacc.at[pl.ds(s * STRIPE + 3072, LAST_REAL - 3072), :],
                    rows.at[0, pl.ds(0, LAST_REAL - 3072), :])
    pltpu.sync_copy(rows.at[0, pl.ds(0, LAST_REAL - 3072), :],
                    z_out.at[pl.ds(out_base + 3072, LAST_REAL - 3072), :])


BLK = 2000
GRID = N // BLK


def _dot(a, b):
  return lax.dot_general(a, b, (((1,), (0,)), ((), ())),
                         preferred_element_type=jnp.float32)


def _in_matmul_body(x_ref, w_ref, deg_ref, o_ref):
  dinv = lax.rsqrt(deg_ref[...] + 1.0)
  o_ref[...] = dinv * _dot(x_ref[...], w_ref[...])


def _pre_block(z_ref, y_ref, deg_ref, b_ref):
  dinv = lax.rsqrt(deg_ref[...] + 1.0)
  return dinv, dinv * (z_ref[...] + y_ref[...]) + b_ref[...]


def _norm_block(pre, acc, g_ref, be_ref):
  st = acc[...]
  mean = st[0:1, :] * (1.0 / N)
  var = st[1:2, :] * (1.0 / N) - mean * mean
  a = g_ref[...] * lax.rsqrt(var + 1e-5)
  cc = be_ref[...] - a * mean
  return jnp.maximum(a * pre + cc, 0.0)


def _fuse_body(z_ref, y_ref, deg_ref, b_ref, g_ref, be_ref, w_ref, o_ref,
               acc):
  ph = pl.program_id(0)
  i = pl.program_id(1)
  dinv, pre = _pre_block(z_ref, y_ref, deg_ref, b_ref)

  @pl.when(ph == 0)
  def _():
    part = jnp.concatenate(
        [jnp.sum(pre, 0, keepdims=True),
         jnp.sum(pre * pre, 0, keepdims=True)], axis=0)

    @pl.when(i == 0)
    def _():
      acc[...] = part

    @pl.when(i > 0)
    def _():
      acc[...] = acc[...] + part

  @pl.when(ph == 1)
  def _():
    h = _norm_block(pre, acc, g_ref, be_ref)
    o_ref[...] = dinv * _dot(h, w_ref[...])


def _head_body(z_ref, y_ref, deg_ref, b_ref, g_ref, be_ref, w_ref, fcb_ref,
               o_ref, acc):
  ph = pl.program_id(0)
  i = pl.program_id(1)
  _, pre = _pre_block(z_ref, y_ref, deg_ref, b_ref)

  @pl.when(ph == 0)
  def _():
    part = jnp.concatenate(
        [jnp.sum(pre, 0, keepdims=True),
         jnp.sum(pre * pre, 0, keepdims=True)], axis=0)

    @pl.when(i == 0)
    def _():
      acc[...] = part

    @pl.when(i > 0)
    def _():
      acc[...] = acc[...] + part

  @pl.when(ph == 1)
  def _():
    h = _norm_block(pre, acc, g_ref, be_ref)
    o_ref[...] = _dot(h, w_ref[...]) + fcb_ref[...]


def _row_spec(cols):
  return pl.BlockSpec((BLK, cols), lambda i: (i, 0))


def _row_spec2(cols):
  return pl.BlockSpec((BLK, cols), lambda p, i: (i, 0))


def _full_spec(shape):
  return pl.BlockSpec(shape, lambda i: tuple(0 for _ in shape))


def _full_spec2(shape):
  return pl.BlockSpec(shape, lambda p, i: tuple(0 for _ in shape))


def _in_matmul(x, w, deg_col):
  return pl.pallas_call(
      _in_matmul_body,
      grid=(GRID,),
      in_specs=[_row_spec(x.shape[1]), _full_spec(w.shape), _row_spec(1)],
      out_specs=_row_spec(HID),
      out_shape=jax.ShapeDtypeStruct((N, HID), jnp.float32),
  )(x, w, deg_col)


def _fuse(z, y, deg_col, b2, g2, be2, wn):
  return pl.pallas_call(
      _fuse_body,
      grid=(2, GRID),
      in_specs=[_row_spec2(HID), _row_spec2(HID), _row_spec2(1),
                _full_spec2((1, HID)), _full_spec2((1, HID)),
                _full_spec2((1, HID)), _full_spec2((HID, HID))],
      out_specs=_row_spec2(HID),
      out_shape=jax.ShapeDtypeStruct((N, HID), jnp.float32),
      scratch_shapes=[pltpu.VMEM((2, HID), jnp.float32)],
  )(z, y, deg_col, b2, g2, be2, wn)


def _head(z, y, deg_col, b2, g2, be2, fcw, fcb2):
  return pl.pallas_call(
      _head_body,
      grid=(2, GRID),
      in_specs=[_row_spec2(HID), _row_spec2(HID), _row_spec2(1),
                _full_spec2((1, HID)), _full_spec2((1, HID)),
                _full_spec2((1, HID)), _full_spec2((HID, 1)),
                _full_spec2((1, 1))],
      out_specs=_row_spec2(1),
      out_shape=jax.ShapeDtypeStruct((N, 1), jnp.float32),
      scratch_shapes=[pltpu.VMEM((2, HID), jnp.float32)],
  )(z, y, deg_col, b2, g2, be2, fcw, fcb2)


def kernel(x, edge_index, edge_attr, W1, b1, g1, be1, W2, b2, g2, be2, W3, b3,
           g3, be3, fcW, fcb):
  del edge_attr
  src = edge_index[0]
  dst = edge_index[1]
  pad = EP - E
  src3 = jnp.concatenate([src, jnp.zeros((pad,), jnp.int32)]).reshape(EPR, LANE)
  dst3 = jnp.concatenate([dst, jnp.full((pad,), -1, jnp.int32)]
                         ).reshape(EPR, LANE)

  deg, ldst3, lsrc3 = _prep_kernel(dst3, src3)
  deg_col = deg.reshape(N, 1)

  y = _in_matmul(x, W1, deg_col)
  params = [(b1, g1, be1), (b2, g2, be2), (b3, g3, be3)]
  mats = [W2, W3]
  for l in range(3):
    z = _edge_kernel(y, lsrc3, ldst3)
    bb, gg, be_ = params[l]
    b2_ = bb.reshape(1, HID)
    g2_ = gg.reshape(1, HID)
    be2_ = be_.reshape(1, HID)
    if l < 2:
      y = _fuse(z, y, deg_col, b2_, g2_, be2_, mats[l])
    else:
      out = _head(z, y, deg_col, b2_, g2_, be2_, fcW, fcb.reshape(1, 1))
  return out
